# Initial kernel scaffold; baseline (speedup 1.0000x reference)
import jax
import jax.numpy as jnp
from jax.experimental import pallas as pl


def kernel(x, label, edge_index, batch, W1l, b1, W1r, W2l, b2, W2r, Wfc, bfc, Wa, ba, Wb, bb, Wc, bc, Wcls, bcls):
    raise NotImplementedError("write your pallas kernel here")


# speedup vs baseline: 6.6076x; 6.6076x over previous
import functools

import jax
import jax.numpy as jnp
from jax import lax
from jax.experimental import pallas as pl
from jax.experimental.pallas import tpu as pltpu
from jax.experimental.pallas import tpu_sc as plsc

_N = 10000
_E = 320000
_DIN = 128
_HID = 256
_DOUT = 512
_DATT = 256
_NCLS = 2

_NC = 2
_NS = 16
_NW = _NC * _NS
_EPW = _E // _NW
_EPT = _E // _NS
_CH = 80
_RPT = 640
_RPT_LAST = _N - (_NS - 1) * _RPT

_HIGH = lax.Precision.HIGHEST

_SC_SCRATCH = [
    pltpu.VMEM_SHARED((_N, 128), jnp.float32),
    pltpu.VMEM((_CH,), jnp.int32),
    pltpu.VMEM((_CH,), jnp.int32),
    pltpu.VMEM((_CH,), jnp.int32),
    pltpu.VMEM((_CH,), jnp.int32),
    pltpu.VMEM((_CH, 128), jnp.float32),
    pltpu.VMEM((_CH, 128), jnp.float32),
    pltpu.SemaphoreType.DMA,
    pltpu.SemaphoreType.DMA,
    pltpu.SemaphoreType.DMA,
    pltpu.SemaphoreType.DMA,
    pltpu.SemaphoreType.DMA,
    pltpu.SemaphoreType.DMA,
]


def _sc_common(src_h, dst_h, z128_h, acc_s,
               ixs0, ixs1, ixd0, ixd1, rw0, rw1,
               sis0, sis1, sid0, sid1, sg0, sg1):
    c = lax.axis_index("c")
    s = lax.axis_index("s")
    row0 = s * _RPT
    ixs = (ixs0, ixs1)
    ixd = (ixd0, ixd1)
    rws = (rw0, rw1)
    sis = (sis0, sis1)
    sid = (sid0, sid1)
    sg = (sg0, sg1)

    def zero_acc():
        @pl.when(s < _NS - 1)
        def _():
            pltpu.sync_copy(z128_h.at[pl.ds(row0, _RPT)],
                            acc_s.at[pl.ds(row0, _RPT)])

        @pl.when(s == _NS - 1)
        def _():
            pltpu.sync_copy(z128_h.at[pl.ds(row0, _RPT_LAST)],
                            acc_s.at[pl.ds(row0, _RPT_LAST)])

    def drain_acc(dst_hbm):
        @pl.when(s < _NS - 1)
        def _():
            pltpu.sync_copy(acc_s.at[pl.ds(row0, _RPT)],
                            dst_hbm.at[pl.ds(c * _N + row0, _RPT)])

        @pl.when(s == _NS - 1)
        def _():
            pltpu.sync_copy(acc_s.at[pl.ds(row0, _RPT_LAST)],
                            dst_hbm.at[pl.ds(c * _N + row0, _RPT_LAST)])

    def issue_is(base, j, b):
        pltpu.async_copy(src_h.at[pl.ds(base + j * _CH, _CH)], ixs[b], sis[b])

    def issue_id(base, j, b):
        pltpu.async_copy(dst_h.at[pl.ds(base + j * _CH, _CH)], ixd[b], sid[b])

    def wait_is(b):
        pltpu.make_async_copy(src_h.at[pl.ds(0, _CH)], ixs[b], sis[b]).wait()

    def wait_id(b):
        pltpu.make_async_copy(dst_h.at[pl.ds(0, _CH)], ixd[b], sid[b]).wait()

    def edge_phase(table_h, base, nchunk):
        def start_gather(b):
            wait_is(b)
            pltpu.async_copy(table_h.at[ixs[b]], rws[b], sg[b])

        def finish(b):
            pltpu.make_async_copy(z128_h.at[pl.ds(0, _CH)], rws[b],
                                  sg[b]).wait()
            wait_id(b)
            pltpu.sync_copy(rws[b], acc_s.at[ixd[b]], add=True)

        npair = (nchunk - 2) // 2
        rem = nchunk - 2 * npair

        issue_is(base, 0, 0)
        issue_id(base, 0, 0)
        issue_is(base, 1, 1)
        issue_id(base, 1, 1)
        start_gather(0)

        def pair(p, carry):
            j = 2 * p
            start_gather(1)
            finish(0)
            issue_is(base, j + 2, 0)
            issue_id(base, j + 2, 0)
            start_gather(0)
            finish(1)
            issue_is(base, j + 3, 1)
            issue_id(base, j + 3, 1)
            return carry

        lax.fori_loop(0, npair, pair, 0)
        start_gather(1)
        finish(0)
        if rem == 3:
            issue_is(base, nchunk - 1, 0)
            issue_id(base, nchunk - 1, 0)
            start_gather(0)
            finish(1)
            finish(0)
        else:
            finish(1)

    def cnt_phase(ones_v, base, nchunk):
        def cscat(b):
            wait_id(b)
            pltpu.sync_copy(ones_v, acc_s.at[ixd[b]], add=True)

        npair = (nchunk - 2) // 2
        rem = nchunk - 2 * npair

        issue_id(base, 0, 0)
        issue_id(base, 1, 1)

        def cpair(p, carry):
            j = 2 * p
            cscat(0)
            issue_id(base, j + 2, 0)
            cscat(1)
            issue_id(base, j + 3, 1)
            return carry

        lax.fori_loop(0, npair, cpair, 0)
        cscat(0)
        if rem == 3:
            cscat(1)
            issue_id(base, nchunk - 1, 0)
            cscat(0)
        else:
            cscat(1)

    return c, s, zero_acc, drain_acc, edge_phase, cnt_phase


def _make_sc_layer1():
    mesh = plsc.VectorSubcoreMesh(core_axis_name="c", subcore_axis_name="s")
    outs = (jax.ShapeDtypeStruct((_NC * _N, 128), jnp.float32),
            jax.ShapeDtypeStruct((_NC * _N, 128), jnp.float32))
    scratch = _SC_SCRATCH + [pltpu.VMEM((_CH, 128), jnp.float32)]

    def body(src_h, dst_h, table_h, z128_h, ones_h, out_h, cnt_h,
             acc_s, ixs0, ixs1, ixd0, ixd1, rw0, rw1,
             sis0, sis1, sid0, sid1, sg0, sg1, ones_v):
        c, s, zero_acc, drain_acc, edge_phase, cnt_phase = _sc_common(
            src_h, dst_h, z128_h, acc_s, ixs0, ixs1, ixd0, ixd1, rw0, rw1,
            sis0, sis1, sid0, sid1, sg0, sg1)
        base = (c * _NS + s) * _EPW
        nchunk = _EPW // _CH

        pltpu.sync_copy(ones_h, ones_v)
        zero_acc()
        plsc.subcore_barrier()
        cnt_phase(ones_v, base, nchunk)
        plsc.subcore_barrier()
        drain_acc(cnt_h)
        plsc.subcore_barrier()
        zero_acc()
        plsc.subcore_barrier()
        edge_phase(table_h, base, nchunk)
        plsc.subcore_barrier()
        drain_acc(out_h)

    return pl.kernel(body, out_type=outs, mesh=mesh, scratch_types=scratch)


def _make_sc_layer2():
    mesh = plsc.VectorSubcoreMesh(core_axis_name="c", subcore_axis_name="s")
    out = jax.ShapeDtypeStruct((_NC * _N, 128), jnp.float32)

    def body(src_h, dst_h, ta_h, tb_h, z128_h, out_h,
             acc_s, ixs0, ixs1, ixd0, ixd1, rw0, rw1,
             sis0, sis1, sid0, sid1, sg0, sg1):
        c, s, zero_acc, drain_acc, edge_phase, cnt_phase = _sc_common(
            src_h, dst_h, z128_h, acc_s, ixs0, ixs1, ixd0, ixd1, rw0, rw1,
            sis0, sis1, sid0, sid1, sg0, sg1)
        base = s * _EPT
        nchunk = _EPT // _CH

        zero_acc()
        plsc.subcore_barrier()

        @pl.when(c == 0)
        def _():
            edge_phase(ta_h, base, nchunk)

        @pl.when(c == 1)
        def _():
            edge_phase(tb_h, base, nchunk)

        plsc.subcore_barrier()
        drain_acc(out_h)

    return pl.kernel(body, out_type=out, mesh=mesh,
                     scratch_types=list(_SC_SCRATCH))


_BN = 1000
_G = _N // _BN


def _tc1_body(aggp, cntp, x, W1l, b1, W1r, h1a_ref, h1b_ref, cnt_ref):
    cnt = jnp.maximum(cntp[0, :, 0:1] + cntp[1, :, 0:1], 1.0)
    agg = (aggp[0] + aggp[1]) / cnt
    t = (jnp.dot(agg, W1l[...], preferred_element_type=jnp.float32,
                 precision=_HIGH)
         + jnp.dot(x[...], W1r[...], preferred_element_type=jnp.float32,
                   precision=_HIGH)
         + b1[...])
    h1 = jnp.tanh(t)
    h1a_ref[...] = h1[:, :128]
    h1b_ref[...] = h1[:, 128:]
    cnt_ref[...] = cnt


def _tc1(aggp, cntp, x, W1l, b1, W1r):
    return pl.pallas_call(
        _tc1_body,
        grid=(_G,),
        in_specs=[
            pl.BlockSpec((_NC, _BN, 128), lambda i: (0, i, 0)),
            pl.BlockSpec((_NC, _BN, 128), lambda i: (0, i, 0)),
            pl.BlockSpec((_BN, _DIN), lambda i: (i, 0)),
            pl.BlockSpec((_DIN, _HID), lambda i: (0, 0)),
            pl.BlockSpec((1, _HID), lambda i: (0, 0)),
            pl.BlockSpec((_DIN, _HID), lambda i: (0, 0)),
        ],
        out_specs=[
            pl.BlockSpec((_BN, 128), lambda i: (i, 0)),
            pl.BlockSpec((_BN, 128), lambda i: (i, 0)),
            pl.BlockSpec((_BN, 1), lambda i: (i, 0)),
        ],
        out_shape=[
            jax.ShapeDtypeStruct((_N, 128), jnp.float32),
            jax.ShapeDtypeStruct((_N, 128), jnp.float32),
            jax.ShapeDtypeStruct((_N, 1), jnp.float32),
        ],
    )(aggp, cntp, x, W1l, b1, W1r)


def _tc2_body(a2f, cnt, h1a, h1b, W2l, b2, W2r, Wfc, bfc,
              Wa, ba, Wb, bb, Wc, bc, Wcls, bcls,
              araw_ref, logits_ref, yprob_ref, yhat_ref,
              mvec_scr, den_scr):
    i = pl.program_id(0)
    c = cnt[...]
    a2a = a2f[0] / c
    a2b = a2f[1] / c
    agg2 = jnp.concatenate([a2a, a2b], axis=1)
    h1 = jnp.concatenate([h1a[...], h1b[...]], axis=1)
    h2 = (jnp.dot(agg2, W2l[...], preferred_element_type=jnp.float32,
                  precision=_HIGH)
          + jnp.dot(h1, W2r[...], preferred_element_type=jnp.float32,
                    precision=_HIGH)
          + b2[...])
    h = jax.nn.relu(jnp.dot(h2, Wfc[...], preferred_element_type=jnp.float32,
                            precision=_HIGH) + bfc[...])
    a = jnp.tanh(jnp.dot(h, Wa[...], preferred_element_type=jnp.float32,
                         precision=_HIGH) + ba[...])
    g = jax.nn.sigmoid(jnp.dot(h, Wb[...], preferred_element_type=jnp.float32,
                               precision=_HIGH) + bb[...])
    ab = (jnp.dot(a * g, Wc[...], preferred_element_type=jnp.float32,
                  precision=_HIGH) + bc[...])
    araw_ref[...] = ab
    wexp = jnp.exp(ab)
    part = lax.dot_general(wexp, h, (((0,), (0,)), ((), ())),
                           preferred_element_type=jnp.float32,
                           precision=_HIGH)
    psum = jnp.sum(wexp)

    @pl.when(i == 0)
    def _():
        mvec_scr[...] = part
        den_scr[0] = psum

    @pl.when(i > 0)
    def _():
        mvec_scr[...] = mvec_scr[...] + part
        den_scr[0] = den_scr[0] + psum

    @pl.when(i == _G - 1)
    def _():
        M = mvec_scr[...] / den_scr[0]
        logits = (jnp.dot(M, Wcls[...], preferred_element_type=jnp.float32,
                          precision=_HIGH) + bcls[...])
        logits_ref[...] = logits
        mx = jnp.max(logits, axis=1, keepdims=True)
        ex = jnp.exp(logits - mx)
        yprob_ref[...] = ex / jnp.sum(ex, axis=1, keepdims=True)
        yhat_ref[...] = jnp.where(logits[0:1, 1:2] > logits[0:1, 0:1],
                                  1, 0).astype(jnp.int32)


def _tc2(a2f, cnt, h1a, h1b, W2l, b2, W2r, Wfc, bfc,
         Wa, ba, Wb, bb, Wc, bc, Wcls, bcls):
    full = lambda r, c: pl.BlockSpec((r, c), lambda i: (0, 0))
    return pl.pallas_call(
        _tc2_body,
        grid=(_G,),
        in_specs=[
            pl.BlockSpec((_NC, _BN, 128), lambda i: (0, i, 0)),
            pl.BlockSpec((_BN, 1), lambda i: (i, 0)),
            pl.BlockSpec((_BN, 128), lambda i: (i, 0)),
            pl.BlockSpec((_BN, 128), lambda i: (i, 0)),
            full(_HID, _DOUT), full(1, _DOUT), full(_HID, _DOUT),
            full(_DOUT, _DOUT), full(1, _DOUT),
            full(_DOUT, _DATT), full(1, _DATT),
            full(_DOUT, _DATT), full(1, _DATT),
            full(_DATT, 1), full(1, 1),
            full(_DOUT, _NCLS), full(1, _NCLS),
        ],
        out_specs=[
            pl.BlockSpec((_BN, 1), lambda i: (i, 0)),
            pl.BlockSpec((1, _NCLS), lambda i: (0, 0)),
            pl.BlockSpec((1, _NCLS), lambda i: (0, 0)),
            pl.BlockSpec((1, 1), lambda i: (0, 0)),
        ],
        out_shape=[
            jax.ShapeDtypeStruct((_N, 1), jnp.float32),
            jax.ShapeDtypeStruct((1, _NCLS), jnp.float32),
            jax.ShapeDtypeStruct((1, _NCLS), jnp.float32),
            jax.ShapeDtypeStruct((1, 1), jnp.int32),
        ],
        scratch_shapes=[
            pltpu.VMEM((1, _DOUT), jnp.float32),
            pltpu.SMEM((1,), jnp.float32),
        ],
    )(a2f, cnt, h1a, h1b, W2l, b2, W2r, Wfc, bfc,
      Wa, ba, Wb, bb, Wc, bc, Wcls, bcls)


def kernel(x, label, edge_index, batch, W1l, b1, W1r, W2l, b2, W2r,
           Wfc, bfc, Wa, ba, Wb, bb, Wc, bc, Wcls, bcls):
    src = edge_index[0]
    dst = edge_index[1]
    z128 = jnp.zeros((_N, 128), jnp.float32)
    ones = jnp.ones((_CH, 128), jnp.float32)

    agg1p, cntp = _make_sc_layer1()(src, dst, x, z128, ones)
    agg1p = agg1p.reshape(_NC, _N, 128)
    cntp = cntp.reshape(_NC, _N, 128)
    h1a, h1b, cnt = _tc1(agg1p, cntp, x, W1l, b1.reshape(1, _HID), W1r)
    a2f = _make_sc_layer2()(src, dst, h1a, h1b, z128).reshape(_NC, _N, 128)
    araw, logits, yprob, yhat = _tc2(
        a2f, cnt, h1a, h1b,
        W2l, b2.reshape(1, _DOUT), W2r,
        Wfc, bfc.reshape(1, _DOUT),
        Wa, ba.reshape(1, _DATT), Wb, bb.reshape(1, _DATT),
        Wc, bc.reshape(1, 1), Wcls, bcls.reshape(1, _NCLS))
    return (logits, yprob, yhat, araw.reshape(1, _N))


# speedup vs baseline: 7.4591x; 1.1289x over previous
import functools

import jax
import jax.numpy as jnp
from jax import lax
from jax.experimental import pallas as pl
from jax.experimental.pallas import tpu as pltpu
from jax.experimental.pallas import tpu_sc as plsc

_N = 10000
_E = 320000
_DIN = 128
_HID = 256
_DOUT = 512
_DATT = 256
_NCLS = 2

_NC = 2
_NS = 16
_NW = _NC * _NS
_EPW = _E // _NW
_EPT = _E // _NS
_CH = 80
_RPT = 640
_RPT_LAST = _N - (_NS - 1) * _RPT

_HIGH = lax.Precision.HIGHEST

_SC_SCRATCH = [
    pltpu.VMEM_SHARED((_N, 128), jnp.float32),
    pltpu.VMEM((_CH,), jnp.int32),
    pltpu.VMEM((_CH,), jnp.int32),
    pltpu.VMEM((_CH,), jnp.int32),
    pltpu.VMEM((_CH,), jnp.int32),
    pltpu.VMEM((_CH,), jnp.int32),
    pltpu.VMEM((_CH,), jnp.int32),
    pltpu.VMEM((_CH, 128), jnp.float32),
    pltpu.VMEM((_CH, 128), jnp.float32),
    pltpu.VMEM((_CH, 128), jnp.float32),
    pltpu.SemaphoreType.DMA,
    pltpu.SemaphoreType.DMA,
    pltpu.SemaphoreType.DMA,
    pltpu.SemaphoreType.DMA,
    pltpu.SemaphoreType.DMA,
    pltpu.SemaphoreType.DMA,
    pltpu.SemaphoreType.DMA,
    pltpu.SemaphoreType.DMA,
    pltpu.SemaphoreType.DMA,
]


def _sc_common(src_h, dst_h, z128_h, acc_s,
               ixs0, ixs1, ixs2, ixd0, ixd1, ixd2, rw0, rw1, rw2,
               sis0, sis1, sis2, sid0, sid1, sid2, sg0, sg1, sg2):
    c = lax.axis_index("c")
    s = lax.axis_index("s")
    row0 = s * _RPT
    ixs = (ixs0, ixs1, ixs2)
    ixd = (ixd0, ixd1, ixd2)
    rws = (rw0, rw1, rw2)
    sis = (sis0, sis1, sis2)
    sid = (sid0, sid1, sid2)
    sg = (sg0, sg1, sg2)

    def zero_acc():
        @pl.when(s < _NS - 1)
        def _():
            pltpu.sync_copy(z128_h.at[pl.ds(row0, _RPT)],
                            acc_s.at[pl.ds(row0, _RPT)])

        @pl.when(s == _NS - 1)
        def _():
            pltpu.sync_copy(z128_h.at[pl.ds(row0, _RPT_LAST)],
                            acc_s.at[pl.ds(row0, _RPT_LAST)])

    def drain_acc(dst_hbm):
        @pl.when(s < _NS - 1)
        def _():
            pltpu.sync_copy(acc_s.at[pl.ds(row0, _RPT)],
                            dst_hbm.at[pl.ds(c * _N + row0, _RPT)])

        @pl.when(s == _NS - 1)
        def _():
            pltpu.sync_copy(acc_s.at[pl.ds(row0, _RPT_LAST)],
                            dst_hbm.at[pl.ds(c * _N + row0, _RPT_LAST)])

    def issue_is(base, j, b):
        pltpu.async_copy(src_h.at[pl.ds(base + j * _CH, _CH)], ixs[b], sis[b])

    def issue_id(base, j, b):
        pltpu.async_copy(dst_h.at[pl.ds(base + j * _CH, _CH)], ixd[b], sid[b])

    def wait_is(b):
        pltpu.make_async_copy(src_h.at[pl.ds(0, _CH)], ixs[b], sis[b]).wait()

    def wait_id(b):
        pltpu.make_async_copy(dst_h.at[pl.ds(0, _CH)], ixd[b], sid[b]).wait()

    def edge_phase(table_h, base, nchunk):
        def start_gather(b):
            wait_is(b)
            pltpu.async_copy(table_h.at[ixs[b]], rws[b], sg[b])

        def finish(b):
            pltpu.make_async_copy(z128_h.at[pl.ds(0, _CH)], rws[b],
                                  sg[b]).wait()
            wait_id(b)
            pltpu.sync_copy(rws[b], acc_s.at[ixd[b]], add=True)

        ntri = (nchunk - 3) // 3
        rem = nchunk - 3 * ntri

        def iss(j, b):
            issue_is(base, j, b)
            issue_id(base, j, b)

        iss(0, 0)
        iss(1, 1)
        iss(2, 2)
        start_gather(0)

        def tri(p, carry):
            j = 3 * p
            start_gather(1)
            finish(0)
            iss(j + 3, 0)
            start_gather(2)
            finish(1)
            iss(j + 4, 1)
            start_gather(0)
            finish(2)
            iss(j + 5, 2)
            return carry

        lax.fori_loop(0, ntri, tri, 0)
        j0 = 3 * ntri
        start_gather(1)
        finish(0)
        if rem >= 4:
            iss(j0 + 3, 0)
        start_gather(2)
        finish(1)
        if rem == 5:
            iss(j0 + 4, 1)
        if rem >= 4:
            start_gather(0)
        finish(2)
        if rem == 5:
            start_gather(1)
        if rem >= 4:
            finish(0)
        if rem == 5:
            finish(1)

    def cnt_phase(ones_v, base, nchunk):
        def cscat(b):
            wait_id(b)
            pltpu.sync_copy(ones_v, acc_s.at[ixd[b]], add=True)

        npair = (nchunk - 2) // 2
        rem = nchunk - 2 * npair

        issue_id(base, 0, 0)
        issue_id(base, 1, 1)

        def cpair(p, carry):
            j = 2 * p
            cscat(0)
            issue_id(base, j + 2, 0)
            cscat(1)
            issue_id(base, j + 3, 1)
            return carry

        lax.fori_loop(0, npair, cpair, 0)
        cscat(0)
        if rem == 3:
            cscat(1)
            issue_id(base, nchunk - 1, 0)
            cscat(0)
        else:
            cscat(1)

    return c, s, zero_acc, drain_acc, edge_phase, cnt_phase


def _make_sc_layer1():
    mesh = plsc.VectorSubcoreMesh(core_axis_name="c", subcore_axis_name="s")
    outs = (jax.ShapeDtypeStruct((_NC * _N, 128), jnp.float32),
            jax.ShapeDtypeStruct((_NC * _N, 128), jnp.float32))
    scratch = _SC_SCRATCH + [pltpu.VMEM((_CH, 128), jnp.float32)]

    def body(src_h, dst_h, table_h, z128_h, ones_h, out_h, cnt_h,
             acc_s, ixs0, ixs1, ixs2, ixd0, ixd1, ixd2, rw0, rw1, rw2,
             sis0, sis1, sis2, sid0, sid1, sid2, sg0, sg1, sg2, ones_v):
        c, s, zero_acc, drain_acc, edge_phase, cnt_phase = _sc_common(
            src_h, dst_h, z128_h, acc_s, ixs0, ixs1, ixs2, ixd0, ixd1, ixd2,
            rw0, rw1, rw2, sis0, sis1, sis2, sid0, sid1, sid2, sg0, sg1, sg2)
        base = (c * _NS + s) * _EPW
        nchunk = _EPW // _CH

        pltpu.sync_copy(ones_h, ones_v)
        zero_acc()
        plsc.subcore_barrier()
        cnt_phase(ones_v, base, nchunk)
        plsc.subcore_barrier()
        drain_acc(cnt_h)
        plsc.subcore_barrier()
        zero_acc()
        plsc.subcore_barrier()
        edge_phase(table_h, base, nchunk)
        plsc.subcore_barrier()
        drain_acc(out_h)

    return pl.kernel(body, out_type=outs, mesh=mesh, scratch_types=scratch)


def _make_sc_layer2():
    mesh = plsc.VectorSubcoreMesh(core_axis_name="c", subcore_axis_name="s")
    out = jax.ShapeDtypeStruct((_NC * _N, 128), jnp.float32)

    def body(src_h, dst_h, ta_h, tb_h, z128_h, out_h,
             acc_s, ixs0, ixs1, ixs2, ixd0, ixd1, ixd2, rw0, rw1, rw2,
             sis0, sis1, sis2, sid0, sid1, sid2, sg0, sg1, sg2):
        c, s, zero_acc, drain_acc, edge_phase, cnt_phase = _sc_common(
            src_h, dst_h, z128_h, acc_s, ixs0, ixs1, ixs2, ixd0, ixd1, ixd2,
            rw0, rw1, rw2, sis0, sis1, sis2, sid0, sid1, sid2, sg0, sg1, sg2)
        base = s * _EPT
        nchunk = _EPT // _CH

        zero_acc()
        plsc.subcore_barrier()

        @pl.when(c == 0)
        def _():
            edge_phase(ta_h, base, nchunk)

        @pl.when(c == 1)
        def _():
            edge_phase(tb_h, base, nchunk)

        plsc.subcore_barrier()
        drain_acc(out_h)

    return pl.kernel(body, out_type=out, mesh=mesh,
                     scratch_types=list(_SC_SCRATCH))


_BN = 1000
_G = _N // _BN


def _tc1_body(aggp, cntp, x, W1l, b1, W1r, h1a_ref, h1b_ref, cnt_ref):
    cnt = jnp.maximum(cntp[0, :, 0:1] + cntp[1, :, 0:1], 1.0)
    agg = (aggp[0] + aggp[1]) / cnt
    t = (jnp.dot(agg, W1l[...], preferred_element_type=jnp.float32,
                 precision=_HIGH)
         + jnp.dot(x[...], W1r[...], preferred_element_type=jnp.float32,
                   precision=_HIGH)
         + b1[...])
    h1 = jnp.tanh(t)
    h1a_ref[...] = h1[:, :128]
    h1b_ref[...] = h1[:, 128:]
    cnt_ref[...] = cnt


def _tc1(aggp, cntp, x, W1l, b1, W1r):
    return pl.pallas_call(
        _tc1_body,
        grid=(_G,),
        in_specs=[
            pl.BlockSpec((_NC, _BN, 128), lambda i: (0, i, 0)),
            pl.BlockSpec((_NC, _BN, 128), lambda i: (0, i, 0)),
            pl.BlockSpec((_BN, _DIN), lambda i: (i, 0)),
            pl.BlockSpec((_DIN, _HID), lambda i: (0, 0)),
            pl.BlockSpec((1, _HID), lambda i: (0, 0)),
            pl.BlockSpec((_DIN, _HID), lambda i: (0, 0)),
        ],
        out_specs=[
            pl.BlockSpec((_BN, 128), lambda i: (i, 0)),
            pl.BlockSpec((_BN, 128), lambda i: (i, 0)),
            pl.BlockSpec((_BN, 1), lambda i: (i, 0)),
        ],
        out_shape=[
            jax.ShapeDtypeStruct((_N, 128), jnp.float32),
            jax.ShapeDtypeStruct((_N, 128), jnp.float32),
            jax.ShapeDtypeStruct((_N, 1), jnp.float32),
        ],
    )(aggp, cntp, x, W1l, b1, W1r)


def _tc2_body(a2f, cnt, h1a, h1b, W2l, b2, W2r, Wfc, bfc,
              Wa, ba, Wb, bb, Wc, bc, Wcls, bcls,
              araw_ref, logits_ref, yprob_ref, yhat_ref,
              mvec_scr, den_scr):
    i = pl.program_id(0)
    c = cnt[...]
    a2a = a2f[0] / c
    a2b = a2f[1] / c
    agg2 = jnp.concatenate([a2a, a2b], axis=1)
    h1 = jnp.concatenate([h1a[...], h1b[...]], axis=1)
    h2 = (jnp.dot(agg2, W2l[...], preferred_element_type=jnp.float32,
                  precision=_HIGH)
          + jnp.dot(h1, W2r[...], preferred_element_type=jnp.float32,
                    precision=_HIGH)
          + b2[...])
    h = jax.nn.relu(jnp.dot(h2, Wfc[...], preferred_element_type=jnp.float32,
                            precision=_HIGH) + bfc[...])
    a = jnp.tanh(jnp.dot(h, Wa[...], preferred_element_type=jnp.float32,
                         precision=_HIGH) + ba[...])
    g = jax.nn.sigmoid(jnp.dot(h, Wb[...], preferred_element_type=jnp.float32,
                               precision=_HIGH) + bb[...])
    ab = (jnp.dot(a * g, Wc[...], preferred_element_type=jnp.float32,
                  precision=_HIGH) + bc[...])
    araw_ref[...] = ab
    wexp = jnp.exp(ab)
    part = lax.dot_general(wexp, h, (((0,), (0,)), ((), ())),
                           preferred_element_type=jnp.float32,
                           precision=_HIGH)
    psum = jnp.sum(wexp)

    @pl.when(i == 0)
    def _():
        mvec_scr[...] = part
        den_scr[0] = psum

    @pl.when(i > 0)
    def _():
        mvec_scr[...] = mvec_scr[...] + part
        den_scr[0] = den_scr[0] + psum

    @pl.when(i == _G - 1)
    def _():
        M = mvec_scr[...] / den_scr[0]
        logits = (jnp.dot(M, Wcls[...], preferred_element_type=jnp.float32,
                          precision=_HIGH) + bcls[...])
        logits_ref[...] = logits
        mx = jnp.max(logits, axis=1, keepdims=True)
        ex = jnp.exp(logits - mx)
        yprob_ref[...] = ex / jnp.sum(ex, axis=1, keepdims=True)
        yhat_ref[...] = jnp.where(logits[0:1, 1:2] > logits[0:1, 0:1],
                                  1, 0).astype(jnp.int32)


def _tc2(a2f, cnt, h1a, h1b, W2l, b2, W2r, Wfc, bfc,
         Wa, ba, Wb, bb, Wc, bc, Wcls, bcls):
    full = lambda r, c: pl.BlockSpec((r, c), lambda i: (0, 0))
    return pl.pallas_call(
        _tc2_body,
        grid=(_G,),
        in_specs=[
            pl.BlockSpec((_NC, _BN, 128), lambda i: (0, i, 0)),
            pl.BlockSpec((_BN, 1), lambda i: (i, 0)),
            pl.BlockSpec((_BN, 128), lambda i: (i, 0)),
            pl.BlockSpec((_BN, 128), lambda i: (i, 0)),
            full(_HID, _DOUT), full(1, _DOUT), full(_HID, _DOUT),
            full(_DOUT, _DOUT), full(1, _DOUT),
            full(_DOUT, _DATT), full(1, _DATT),
            full(_DOUT, _DATT), full(1, _DATT),
            full(_DATT, 1), full(1, 1),
            full(_DOUT, _NCLS), full(1, _NCLS),
        ],
        out_specs=[
            pl.BlockSpec((_BN, 1), lambda i: (i, 0)),
            pl.BlockSpec((1, _NCLS), lambda i: (0, 0)),
            pl.BlockSpec((1, _NCLS), lambda i: (0, 0)),
            pl.BlockSpec((1, 1), lambda i: (0, 0)),
        ],
        out_shape=[
            jax.ShapeDtypeStruct((_N, 1), jnp.float32),
            jax.ShapeDtypeStruct((1, _NCLS), jnp.float32),
            jax.ShapeDtypeStruct((1, _NCLS), jnp.float32),
            jax.ShapeDtypeStruct((1, 1), jnp.int32),
        ],
        scratch_shapes=[
            pltpu.VMEM((1, _DOUT), jnp.float32),
            pltpu.SMEM((1,), jnp.float32),
        ],
    )(a2f, cnt, h1a, h1b, W2l, b2, W2r, Wfc, bfc,
      Wa, ba, Wb, bb, Wc, bc, Wcls, bcls)


def kernel(x, label, edge_index, batch, W1l, b1, W1r, W2l, b2, W2r,
           Wfc, bfc, Wa, ba, Wb, bb, Wc, bc, Wcls, bcls):
    src = edge_index[0]
    dst = edge_index[1]
    z128 = jnp.zeros((_N, 128), jnp.float32)
    ones = jnp.ones((_CH, 128), jnp.float32)

    agg1p, cntp = _make_sc_layer1()(src, dst, x, z128, ones)
    agg1p = agg1p.reshape(_NC, _N, 128)
    cntp = cntp.reshape(_NC, _N, 128)
    h1a, h1b, cnt = _tc1(agg1p, cntp, x, W1l, b1.reshape(1, _HID), W1r)
    a2f = _make_sc_layer2()(src, dst, h1a, h1b, z128).reshape(_NC, _N, 128)
    araw, logits, yprob, yhat = _tc2(
        a2f, cnt, h1a, h1b,
        W2l, b2.reshape(1, _DOUT), W2r,
        Wfc, bfc.reshape(1, _DOUT),
        Wa, ba.reshape(1, _DATT), Wb, bb.reshape(1, _DATT),
        Wc, bc.reshape(1, 1), Wcls, bcls.reshape(1, _NCLS))
    return (logits, yprob, yhat, araw.reshape(1, _N))
